# Initial kernel scaffold; baseline (speedup 1.0000x reference)
import jax
import jax.numpy as jnp
from jax.experimental import pallas as pl


def kernel(batch_confidence_maps, B):
    raise NotImplementedError("write your pallas kernel here")


# speedup vs baseline: 4.4362x; 4.4362x over previous
import functools

import numpy as np
import jax
import jax.numpy as jnp
from jax.experimental import pallas as pl

_THRESHOLD = 0.5
_K_SIZE = 5
_C_SIGMA = 1.0


def _bf16_round(a):
    return np.asarray(a, dtype=np.float32).astype(jnp.bfloat16).astype(np.float32)


def _gauss_rank_factors(k_size=_K_SIZE, sigma=_C_SIGMA):
    center = k_size // 2
    x, y = np.mgrid[-center:k_size - center, -center:k_size - center]
    g = 1.0 / (2.0 * np.pi * sigma) * np.exp(
        -(np.square(x) + np.square(y)) / (2.0 * np.square(sigma)))
    wb = np.asarray(_bf16_round(g), dtype=np.float64)
    u_mat, s_vals, vt_mat = np.linalg.svd(wb)
    rank = int((s_vals > 1e-12).sum())
    us = [(u_mat[:, r] * s_vals[r]).astype(np.float32) for r in range(rank)]
    vs = [vt_mat[r].astype(np.float32) for r in range(rank)]
    return us, vs


def _band_matrix(n, taps):
    k = len(taps)
    c = k // 2
    m = np.zeros((n, n), dtype=np.float32)
    for d in range(k):
        off = d - c
        idx = np.arange(max(0, -off), min(n, n - off))
        m[idx, idx + off] = taps[d]
    return m


def _fused_kernel(conf_ref, bh_ref, bw_ref, raw_ref, mask_ref, sum_ref, *,
                  ego_stride, H, W, rank):
    pid = pl.program_id(0)
    m = jnp.maximum(conf_ref[:, 0], conf_ref[:, 1])
    s = jax.nn.sigmoid(m).reshape(2 * H, W)
    sb = s.astype(jnp.bfloat16).astype(jnp.float32)
    HH = 2 * H
    xs = jax.lax.dot_general(
        bh_ref[...], sb, (((1,), (0,)), ((), ())),
        precision=jax.lax.Precision.HIGHEST,
        preferred_element_type=jnp.float32)
    v = jax.lax.dot_general(
        xs[0:HH], bw_ref[0:W], (((1,), (0,)), ((), ())),
        precision=jax.lax.Precision.HIGHEST,
        preferred_element_type=jnp.float32)
    for r in range(1, rank):
        v = v + jax.lax.dot_general(
            xs[r * HH:(r + 1) * HH], bw_ref[r * W:(r + 1) * W],
            (((1,), (0,)), ((), ())),
            precision=jax.lax.Precision.HIGHEST,
            preferred_element_type=jnp.float32)
    v = v.reshape(2, H, W)
    raw_ref[...] = v
    mask = jnp.where(v > _THRESHOLD, 1.0, 0.0).astype(jnp.float32)
    sum_ref[0] = jnp.sum(mask)[None, None]
    is_ego = (pid % ego_stride) == 0
    mask_ref[0] = jnp.where(is_ego, jnp.ones_like(mask[0]), mask[0])
    mask_ref[1] = mask[1]


def kernel(batch_confidence_maps, B):
    conf = batch_confidence_maps
    Bn, L, A, H, W = conf.shape
    n_maps = Bn * L
    flat = conf.reshape(n_maps, A, H, W)

    us, vs = _gauss_rank_factors()
    rank = len(us)
    bh_blocks = []
    bw_blocks = []
    for u, v in zip(us, vs):
        bh1 = _band_matrix(H, u)
        z = np.zeros_like(bh1)
        bh_blocks.append(np.block([[bh1, z], [z, bh1]]))
        bw_blocks.append(_band_matrix(W, v).T)
    bh = jnp.asarray(np.concatenate(bh_blocks, axis=0))
    bw = jnp.asarray(np.concatenate(bw_blocks, axis=0))

    n_steps = n_maps // 2
    raw, mask, sums = pl.pallas_call(
        functools.partial(_fused_kernel, ego_stride=L // 2, H=H, W=W,
                          rank=rank),
        grid=(n_steps,),
        in_specs=[
            pl.BlockSpec((2, A, H, W), lambda i: (i, 0, 0, 0)),
            pl.BlockSpec((rank * 2 * H, 2 * H), lambda i: (0, 0)),
            pl.BlockSpec((rank * W, W), lambda i: (0, 0)),
        ],
        out_specs=[
            pl.BlockSpec((2, H, W), lambda i: (i, 0, 0)),
            pl.BlockSpec((2, H, W), lambda i: (i, 0, 0)),
            pl.BlockSpec((1, 1, 1), lambda i: (i, 0, 0)),
        ],
        out_shape=[
            jax.ShapeDtypeStruct((n_maps, H, W), jnp.float32),
            jax.ShapeDtypeStruct((n_maps, H, W), jnp.float32),
            jax.ShapeDtypeStruct((n_steps, 1, 1), jnp.float32),
        ],
    )(flat, bh, bw)

    communication_masks = mask.reshape(n_maps, 1, H, W)
    raw_out = raw.reshape(Bn, L, 1, H, W)
    total = jnp.sum(sums)
    communication_rate = total / jnp.float32(L * H * W) / jnp.float32(Bn)
    batch_check = (jnp.asarray(B) - Bn) * 0
    communication_rate = communication_rate + batch_check.astype(jnp.float32)
    return (communication_masks, communication_rate, raw_out)


# speedup vs baseline: 7.2868x; 1.6426x over previous
import functools

import numpy as np
import jax
import jax.numpy as jnp
from jax.experimental import pallas as pl

_THRESHOLD = 0.5
_K_SIZE = 5
_C_SIGMA = 1.0


def _bf16_round(a):
    return np.asarray(a, dtype=np.float32).astype(jnp.bfloat16).astype(np.float32)


def _gauss_rank_factors(k_size=_K_SIZE, sigma=_C_SIGMA):
    center = k_size // 2
    x, y = np.mgrid[-center:k_size - center, -center:k_size - center]
    g = 1.0 / (2.0 * np.pi * sigma) * np.exp(
        -(np.square(x) + np.square(y)) / (2.0 * np.square(sigma)))
    wb = np.asarray(_bf16_round(g), dtype=np.float64)
    u_mat, s_vals, vt_mat = np.linalg.svd(wb)
    rank = int((s_vals > 1e-12).sum())
    us = [(u_mat[:, r] * s_vals[r]).astype(np.float32) for r in range(rank)]
    vs = [vt_mat[r].astype(np.float32) for r in range(rank)]
    return us, vs


def _band_matrix(n, taps):
    k = len(taps)
    c = k // 2
    m = np.zeros((n, n), dtype=np.float32)
    for d in range(k):
        off = d - c
        idx = np.arange(max(0, -off), min(n, n - off))
        m[idx, idx + off] = taps[d]
    return m


def _split_bf16(a32):
    hi = np.asarray(a32.astype(jnp.bfloat16))
    r1 = a32 - hi.astype(np.float32)
    lo = np.asarray(r1.astype(jnp.bfloat16))
    r2 = r1 - lo.astype(np.float32)
    lo2 = np.asarray(r2.astype(jnp.bfloat16))
    return hi, lo, lo2


def _dot16(a, b):
    return jax.lax.dot_general(
        a, b, (((1,), (0,)), ((), ())),
        preferred_element_type=jnp.float32)


def _fused_kernel(conf_ref, bh_hi_ref, bh_lo_ref, bh1_lo2_ref,
                  bw_hi_ref, bw_lo_ref, bw1_lo2_ref,
                  raw_ref, mask_ref, sum_ref, *, ego_stride, H, W, rank):
    pid = pl.program_id(0)
    m = jnp.maximum(conf_ref[:, 0], conf_ref[:, 1])
    s = jax.nn.sigmoid(m).reshape(2 * H, W)
    sb = s.astype(jnp.bfloat16)
    HH = 2 * H
    xs = _dot16(bh_hi_ref[...], sb) + _dot16(bh_lo_ref[...], sb)
    x1 = xs[0:HH] + _dot16(bh1_lo2_ref[...], sb)
    v = None
    for r in range(rank):
        a = x1 if r == 0 else xs[r * HH:(r + 1) * HH]
        a_hi = a.astype(jnp.bfloat16)
        a_lo = (a - a_hi.astype(jnp.float32)).astype(jnp.bfloat16)
        bw_hi = bw_hi_ref[r * W:(r + 1) * W]
        bw_lo = bw_lo_ref[r * W:(r + 1) * W]
        t = _dot16(a_hi, bw_hi) + _dot16(a_hi, bw_lo) + _dot16(a_lo, bw_hi)
        if r == 0:
            t = t + _dot16(a_lo, bw_lo) + _dot16(a_hi, bw1_lo2_ref[...])
        v = t if v is None else v + t
    v = v.reshape(2, H, W)
    raw_ref[...] = v
    mask = jnp.where(v > _THRESHOLD, 1.0, 0.0).astype(jnp.float32)
    sum_ref[0] = jnp.sum(mask)[None, None]
    is_ego = (pid % ego_stride) == 0
    mask_ref[0] = jnp.where(is_ego, jnp.ones_like(mask[0]), mask[0])
    mask_ref[1] = mask[1]


def kernel(batch_confidence_maps, B):
    conf = batch_confidence_maps
    Bn, L, A, H, W = conf.shape
    n_maps = Bn * L
    flat = conf.reshape(n_maps, A, H, W)

    us, vs = _gauss_rank_factors()
    rank = len(us)
    bh_blocks = []
    bw_blocks = []
    for u, v in zip(us, vs):
        bh1 = _band_matrix(H, u)
        z = np.zeros_like(bh1)
        bh_blocks.append(np.block([[bh1, z], [z, bh1]]))
        bw_blocks.append(_band_matrix(W, v).T)
    bh = np.concatenate(bh_blocks, axis=0)
    bw = np.concatenate(bw_blocks, axis=0)

    bh_hi, bh_lo, bh_lo2 = _split_bf16(bh)
    bw_hi, bw_lo, bw_lo2 = _split_bf16(bw)
    bh1_lo2 = jnp.asarray(bh_lo2[0:2 * H])
    bw1_lo2 = jnp.asarray(bw_lo2[0:W])

    n_steps = n_maps // 2
    const_spec = [
        pl.BlockSpec((rank * 2 * H, 2 * H), lambda i: (0, 0)),
        pl.BlockSpec((rank * 2 * H, 2 * H), lambda i: (0, 0)),
        pl.BlockSpec((2 * H, 2 * H), lambda i: (0, 0)),
        pl.BlockSpec((rank * W, W), lambda i: (0, 0)),
        pl.BlockSpec((rank * W, W), lambda i: (0, 0)),
        pl.BlockSpec((W, W), lambda i: (0, 0)),
    ]
    raw, mask, sums = pl.pallas_call(
        functools.partial(_fused_kernel, ego_stride=L // 2, H=H, W=W,
                          rank=rank),
        grid=(n_steps,),
        in_specs=[pl.BlockSpec((2, A, H, W), lambda i: (i, 0, 0, 0))]
                 + const_spec,
        out_specs=[
            pl.BlockSpec((2, H, W), lambda i: (i, 0, 0)),
            pl.BlockSpec((2, H, W), lambda i: (i, 0, 0)),
            pl.BlockSpec((1, 1, 1), lambda i: (i, 0, 0)),
        ],
        out_shape=[
            jax.ShapeDtypeStruct((n_maps, H, W), jnp.float32),
            jax.ShapeDtypeStruct((n_maps, H, W), jnp.float32),
            jax.ShapeDtypeStruct((n_steps, 1, 1), jnp.float32),
        ],
    )(flat, jnp.asarray(bh_hi), jnp.asarray(bh_lo), bh1_lo2,
      jnp.asarray(bw_hi), jnp.asarray(bw_lo), bw1_lo2)

    communication_masks = mask.reshape(n_maps, 1, H, W)
    raw_out = raw.reshape(Bn, L, 1, H, W)
    total = jnp.sum(sums)
    communication_rate = total / jnp.float32(L * H * W) / jnp.float32(Bn)
    batch_check = (jnp.asarray(B) - Bn) * 0
    communication_rate = communication_rate + batch_check.astype(jnp.float32)
    return (communication_masks, communication_rate, raw_out)


# speedup vs baseline: 8.4329x; 1.1573x over previous
import functools

import numpy as np
import jax
import jax.numpy as jnp
from jax.experimental import pallas as pl

_THRESHOLD = 0.5
_K_SIZE = 5
_C_SIGMA = 1.0


def _bf16_round(a):
    return np.asarray(a, dtype=np.float32).astype(jnp.bfloat16).astype(np.float32)


def _gauss_rank_factors(k_size=_K_SIZE, sigma=_C_SIGMA):
    center = k_size // 2
    x, y = np.mgrid[-center:k_size - center, -center:k_size - center]
    g = 1.0 / (2.0 * np.pi * sigma) * np.exp(
        -(np.square(x) + np.square(y)) / (2.0 * np.square(sigma)))
    wb = np.asarray(_bf16_round(g), dtype=np.float64)
    u_mat, s_vals, vt_mat = np.linalg.svd(wb)
    rank = int((s_vals > 1e-12).sum())
    us = [(u_mat[:, r] * s_vals[r]).astype(np.float32) for r in range(rank)]
    vs = [vt_mat[r].astype(np.float32) for r in range(rank)]
    return us, vs


def _band_matrix(n, taps):
    k = len(taps)
    c = k // 2
    m = np.zeros((n, n), dtype=np.float32)
    for d in range(k):
        off = d - c
        idx = np.arange(max(0, -off), min(n, n - off))
        m[idx, idx + off] = taps[d]
    return m


def _split_bf16(a32):
    hi = np.asarray(a32.astype(jnp.bfloat16))
    r1 = a32 - hi.astype(np.float32)
    lo = np.asarray(r1.astype(jnp.bfloat16))
    r2 = r1 - lo.astype(np.float32)
    lo2 = np.asarray(r2.astype(jnp.bfloat16))
    return hi, lo, lo2


def _dot16(a, b):
    return jax.lax.dot_general(
        a, b, (((1,), (0,)), ((), ())),
        preferred_element_type=jnp.float32)


def _smooth_pair(sb, bh_hi_ref, bh_lo_ref, bh1_lo2_ref,
                 bw_hi_ref, bw_lo_ref, bw1_lo2_ref, HH, W, rank):
    xs = _dot16(bh_hi_ref[...], sb) + _dot16(bh_lo_ref[...], sb)
    x1 = xs[0:HH] + _dot16(bh1_lo2_ref[...], sb)
    v = None
    for r in range(rank):
        a = x1 if r == 0 else xs[r * HH:(r + 1) * HH]
        a_hi = a.astype(jnp.bfloat16)
        a_lo = (a - a_hi.astype(jnp.float32)).astype(jnp.bfloat16)
        bw_hi = bw_hi_ref[r * W:(r + 1) * W]
        bw_lo = bw_lo_ref[r * W:(r + 1) * W]
        t = _dot16(a_hi, bw_hi) + _dot16(a_hi, bw_lo) + _dot16(a_lo, bw_hi)
        if r == 0:
            t = t + _dot16(a_lo, bw_lo) + _dot16(a_hi, bw1_lo2_ref[...])
        v = t if v is None else v + t
    return v


def _fused_kernel(conf_ref, bh_hi_ref, bh_lo_ref, bh1_lo2_ref,
                  bw_hi_ref, bw_lo_ref, bw1_lo2_ref,
                  raw_ref, mask_ref, sum_ref, *, ego_stride, H, W, rank,
                  pairs):
    pid = pl.program_id(0)
    HH = 2 * H
    total = None
    for p in range(pairs):
        m = jnp.maximum(conf_ref[2 * p:2 * p + 2, 0],
                        conf_ref[2 * p:2 * p + 2, 1])
        s = jax.nn.sigmoid(m).reshape(HH, W)
        sb = s.astype(jnp.bfloat16)
        v = _smooth_pair(sb, bh_hi_ref, bh_lo_ref, bh1_lo2_ref,
                         bw_hi_ref, bw_lo_ref, bw1_lo2_ref, HH, W, rank)
        v = v.reshape(2, H, W)
        raw_ref[2 * p:2 * p + 2] = v
        mask = jnp.where(v > _THRESHOLD, 1.0, 0.0).astype(jnp.float32)
        t = jnp.sum(mask)
        total = t if total is None else total + t
        is_ego = ((pairs * pid + p) % ego_stride) == 0
        mask_ref[2 * p] = jnp.where(is_ego, jnp.ones_like(mask[0]), mask[0])
        mask_ref[2 * p + 1] = mask[1]
    sum_ref[0] = total[None, None]


def kernel(batch_confidence_maps, B):
    conf = batch_confidence_maps
    Bn, L, A, H, W = conf.shape
    n_maps = Bn * L
    flat = conf.reshape(n_maps, A, H, W)

    us, vs = _gauss_rank_factors()
    rank = len(us)
    bh_blocks = []
    bw_blocks = []
    for u, v in zip(us, vs):
        bh1 = _band_matrix(H, u)
        z = np.zeros_like(bh1)
        bh_blocks.append(np.block([[bh1, z], [z, bh1]]))
        bw_blocks.append(_band_matrix(W, v).T)
    bh = np.concatenate(bh_blocks, axis=0)
    bw = np.concatenate(bw_blocks, axis=0)

    bh_hi, bh_lo, bh_lo2 = _split_bf16(bh)
    bw_hi, bw_lo, bw_lo2 = _split_bf16(bw)
    bh1_lo2 = jnp.asarray(bh_lo2[0:2 * H])
    bw1_lo2 = jnp.asarray(bw_lo2[0:W])

    pairs = 2
    n_steps = n_maps // (2 * pairs)
    const_spec = [
        pl.BlockSpec((rank * 2 * H, 2 * H), lambda i: (0, 0)),
        pl.BlockSpec((rank * 2 * H, 2 * H), lambda i: (0, 0)),
        pl.BlockSpec((2 * H, 2 * H), lambda i: (0, 0)),
        pl.BlockSpec((rank * W, W), lambda i: (0, 0)),
        pl.BlockSpec((rank * W, W), lambda i: (0, 0)),
        pl.BlockSpec((W, W), lambda i: (0, 0)),
    ]
    raw, mask, sums = pl.pallas_call(
        functools.partial(_fused_kernel, ego_stride=L // 2, H=H, W=W,
                          rank=rank, pairs=pairs),
        grid=(n_steps,),
        in_specs=[pl.BlockSpec((2 * pairs, A, H, W), lambda i: (i, 0, 0, 0))]
                 + const_spec,
        out_specs=[
            pl.BlockSpec((2 * pairs, H, W), lambda i: (i, 0, 0)),
            pl.BlockSpec((2 * pairs, H, W), lambda i: (i, 0, 0)),
            pl.BlockSpec((1, 1, 1), lambda i: (i, 0, 0)),
        ],
        out_shape=[
            jax.ShapeDtypeStruct((n_maps, H, W), jnp.float32),
            jax.ShapeDtypeStruct((n_maps, H, W), jnp.float32),
            jax.ShapeDtypeStruct((n_steps, 1, 1), jnp.float32),
        ],
    )(flat, jnp.asarray(bh_hi), jnp.asarray(bh_lo), bh1_lo2,
      jnp.asarray(bw_hi), jnp.asarray(bw_lo), bw1_lo2)

    communication_masks = mask.reshape(n_maps, 1, H, W)
    raw_out = raw.reshape(Bn, L, 1, H, W)
    total = jnp.sum(sums)
    communication_rate = total / jnp.float32(L * H * W) / jnp.float32(Bn)
    batch_check = (jnp.asarray(B) - Bn) * 0
    communication_rate = communication_rate + batch_check.astype(jnp.float32)
    return (communication_masks, communication_rate, raw_out)


# speedup vs baseline: 10.3055x; 1.2221x over previous
import functools

import numpy as np
import jax
import jax.numpy as jnp
from jax.experimental import pallas as pl

_THRESHOLD = 0.5
_K_SIZE = 5
_C_SIGMA = 1.0


def _bf16_round(a):
    return np.asarray(a, dtype=np.float32).astype(jnp.bfloat16).astype(np.float32)


def _gauss_rank_factors(k_size=_K_SIZE, sigma=_C_SIGMA):
    center = k_size // 2
    x, y = np.mgrid[-center:k_size - center, -center:k_size - center]
    g = 1.0 / (2.0 * np.pi * sigma) * np.exp(
        -(np.square(x) + np.square(y)) / (2.0 * np.square(sigma)))
    wb = np.asarray(_bf16_round(g), dtype=np.float64)
    u_mat, s_vals, vt_mat = np.linalg.svd(wb)
    rank = int((s_vals > 1e-12).sum())
    us = [(u_mat[:, r] * s_vals[r]).astype(np.float32) for r in range(rank)]
    vs = [vt_mat[r].astype(np.float32) for r in range(rank)]
    return us, vs


def _band_matrix(n, taps):
    k = len(taps)
    c = k // 2
    m = np.zeros((n, n), dtype=np.float32)
    for d in range(k):
        off = d - c
        idx = np.arange(max(0, -off), min(n, n - off))
        m[idx, idx + off] = taps[d]
    return m


def _split_bf16(a32):
    hi = np.asarray(a32.astype(jnp.bfloat16))
    r1 = a32 - hi.astype(np.float32)
    lo = np.asarray(r1.astype(jnp.bfloat16))
    r2 = r1 - lo.astype(np.float32)
    lo2 = np.asarray(r2.astype(jnp.bfloat16))
    return hi, lo, lo2


def _dot16(a, b):
    return jax.lax.dot_general(
        a, b, (((1,), (0,)), ((), ())),
        preferred_element_type=jnp.float32)


def _smooth_pair(sb, bh_hi_ref, bh_lo_ref, bh1_lo2_ref,
                 bw_hi_ref, bw_lo_ref, bw1_lo2_ref, HH, W, rank):
    xs = _dot16(bh_hi_ref[...], sb) + _dot16(bh_lo_ref[...], sb)
    x1 = xs[0:HH] + _dot16(bh1_lo2_ref[...], sb)
    v = None
    for r in range(rank):
        a = x1 if r == 0 else xs[r * HH:(r + 1) * HH]
        a_hi = a.astype(jnp.bfloat16)
        a_lo = (a - a_hi.astype(jnp.float32)).astype(jnp.bfloat16)
        bw_hi = bw_hi_ref[r * W:(r + 1) * W]
        bw_lo = bw_lo_ref[r * W:(r + 1) * W]
        t = _dot16(a_hi, bw_hi) + _dot16(a_hi, bw_lo) + _dot16(a_lo, bw_hi)
        if r == 0:
            t = t + _dot16(a_lo, bw_lo) + _dot16(a_hi, bw1_lo2_ref[...])
        v = t if v is None else v + t
    return v


def _fused_kernel(conf_ref, bh_hi_ref, bh_lo_ref, bh1_lo2_ref,
                  bw_hi_ref, bw_lo_ref, bw1_lo2_ref,
                  raw_ref, mask_ref, sum_ref, *, ego_stride, H, W, rank,
                  pairs):
    pid = pl.program_id(0)
    HH = 2 * H
    total = None
    for p in range(pairs):
        m = jnp.maximum(conf_ref[2 * p:2 * p + 2, 0],
                        conf_ref[2 * p:2 * p + 2, 1])
        s = jax.nn.sigmoid(m).reshape(HH, W)
        sb = s.astype(jnp.bfloat16)
        v = _smooth_pair(sb, bh_hi_ref, bh_lo_ref, bh1_lo2_ref,
                         bw_hi_ref, bw_lo_ref, bw1_lo2_ref, HH, W, rank)
        v = v.reshape(2, H, W)
        raw_ref[2 * p:2 * p + 2] = v
        mask = jnp.where(v > _THRESHOLD, 1.0, 0.0).astype(jnp.float32)
        t = jnp.sum(mask)
        total = t if total is None else total + t
        is_ego = ((pairs * pid + p) % ego_stride) == 0
        mask_ref[2 * p] = jnp.where(is_ego, jnp.ones_like(mask[0]), mask[0])
        mask_ref[2 * p + 1] = mask[1]
    sum_ref[0] = total[None, None]


def kernel(batch_confidence_maps, B):
    conf = batch_confidence_maps
    Bn, L, A, H, W = conf.shape
    n_maps = Bn * L
    flat = conf.reshape(n_maps, A, H, W)

    us, vs = _gauss_rank_factors()
    rank = len(us)
    bh_blocks = []
    bw_blocks = []
    for u, v in zip(us, vs):
        bh1 = _band_matrix(H, u)
        z = np.zeros_like(bh1)
        bh_blocks.append(np.block([[bh1, z], [z, bh1]]))
        bw_blocks.append(_band_matrix(W, v).T)
    bh = np.concatenate(bh_blocks, axis=0)
    bw = np.concatenate(bw_blocks, axis=0)

    bh_hi, bh_lo, bh_lo2 = _split_bf16(bh)
    bw_hi, bw_lo, bw_lo2 = _split_bf16(bw)
    bh1_lo2 = jnp.asarray(bh_lo2[0:2 * H])
    bw1_lo2 = jnp.asarray(bw_lo2[0:W])

    pairs = 4
    n_steps = n_maps // (2 * pairs)
    const_spec = [
        pl.BlockSpec((rank * 2 * H, 2 * H), lambda i: (0, 0)),
        pl.BlockSpec((rank * 2 * H, 2 * H), lambda i: (0, 0)),
        pl.BlockSpec((2 * H, 2 * H), lambda i: (0, 0)),
        pl.BlockSpec((rank * W, W), lambda i: (0, 0)),
        pl.BlockSpec((rank * W, W), lambda i: (0, 0)),
        pl.BlockSpec((W, W), lambda i: (0, 0)),
    ]
    raw, mask, sums = pl.pallas_call(
        functools.partial(_fused_kernel, ego_stride=L // 2, H=H, W=W,
                          rank=rank, pairs=pairs),
        grid=(n_steps,),
        in_specs=[pl.BlockSpec((2 * pairs, A, H, W), lambda i: (i, 0, 0, 0))]
                 + const_spec,
        out_specs=[
            pl.BlockSpec((2 * pairs, H, W), lambda i: (i, 0, 0)),
            pl.BlockSpec((2 * pairs, H, W), lambda i: (i, 0, 0)),
            pl.BlockSpec((1, 1, 1), lambda i: (i, 0, 0)),
        ],
        out_shape=[
            jax.ShapeDtypeStruct((n_maps, H, W), jnp.float32),
            jax.ShapeDtypeStruct((n_maps, H, W), jnp.float32),
            jax.ShapeDtypeStruct((n_steps, 1, 1), jnp.float32),
        ],
    )(flat, jnp.asarray(bh_hi), jnp.asarray(bh_lo), bh1_lo2,
      jnp.asarray(bw_hi), jnp.asarray(bw_lo), bw1_lo2)

    communication_masks = mask.reshape(n_maps, 1, H, W)
    raw_out = raw.reshape(Bn, L, 1, H, W)
    total = jnp.sum(sums)
    communication_rate = total / jnp.float32(L * H * W) / jnp.float32(Bn)
    batch_check = (jnp.asarray(B) - Bn) * 0
    communication_rate = communication_rate + batch_check.astype(jnp.float32)
    return (communication_masks, communication_rate, raw_out)


# speedup vs baseline: 13.4638x; 1.3065x over previous
import functools

import numpy as np
import jax
import jax.numpy as jnp
from jax.experimental import pallas as pl

_THRESHOLD = 0.5
_K_SIZE = 5
_C_SIGMA = 1.0


def _bf16_round(a):
    return np.asarray(a, dtype=np.float32).astype(jnp.bfloat16).astype(np.float32)


def _gauss_rank_factors(k_size=_K_SIZE, sigma=_C_SIGMA):
    center = k_size // 2
    x, y = np.mgrid[-center:k_size - center, -center:k_size - center]
    g = 1.0 / (2.0 * np.pi * sigma) * np.exp(
        -(np.square(x) + np.square(y)) / (2.0 * np.square(sigma)))
    wb = np.asarray(_bf16_round(g), dtype=np.float64)
    u_mat, s_vals, vt_mat = np.linalg.svd(wb)
    rank = int((s_vals > 1e-12).sum())
    us = [(u_mat[:, r] * s_vals[r]).astype(np.float32) for r in range(rank)]
    vs = [vt_mat[r].astype(np.float32) for r in range(rank)]
    return us, vs


def _band_matrix(n, taps):
    k = len(taps)
    c = k // 2
    m = np.zeros((n, n), dtype=np.float32)
    for d in range(k):
        off = d - c
        idx = np.arange(max(0, -off), min(n, n - off))
        m[idx, idx + off] = taps[d]
    return m


def _split_bf16(a32):
    hi = np.asarray(a32.astype(jnp.bfloat16))
    r1 = a32 - hi.astype(np.float32)
    lo = np.asarray(r1.astype(jnp.bfloat16))
    r2 = r1 - lo.astype(np.float32)
    lo2 = np.asarray(r2.astype(jnp.bfloat16))
    return hi, lo, lo2


def _dot16(a, b):
    return jax.lax.dot_general(
        a, b, (((1,), (0,)), ((), ())),
        preferred_element_type=jnp.float32)


def _smooth_pair(sb, bh1_hi_ref, bh1_lo_ref, bh1_lo2_ref, bh23_hi_ref,
                 bw1_hi_ref, bw1_lo_ref, bw1_lo2_ref, bw23_hi_ref,
                 HH, W, rank):
    x1 = (_dot16(bh1_hi_ref[...], sb) + _dot16(bh1_lo_ref[...], sb)
          + _dot16(bh1_lo2_ref[...], sb))
    a_hi = x1.astype(jnp.bfloat16)
    a_lo = (x1 - a_hi.astype(jnp.float32)).astype(jnp.bfloat16)
    v = (_dot16(a_hi, bw1_hi_ref[...]) + _dot16(a_hi, bw1_lo_ref[...])
         + _dot16(a_lo, bw1_hi_ref[...]) + _dot16(a_lo, bw1_lo_ref[...])
         + _dot16(a_hi, bw1_lo2_ref[...]))
    xs23 = _dot16(bh23_hi_ref[...], sb)
    for r in range(rank - 1):
        v = v + _dot16(xs23[r * HH:(r + 1) * HH].astype(jnp.bfloat16),
                       bw23_hi_ref[r * W:(r + 1) * W])
    return v


def _fused_kernel(conf_ref, bh1_hi_ref, bh1_lo_ref, bh1_lo2_ref, bh23_hi_ref,
                  bw1_hi_ref, bw1_lo_ref, bw1_lo2_ref, bw23_hi_ref,
                  raw_ref, mask_ref, sum_ref, *, ego_stride, H, W, rank,
                  pairs):
    pid = pl.program_id(0)
    HH = 2 * H
    total = None
    for p in range(pairs):
        m = jnp.maximum(conf_ref[2 * p:2 * p + 2, 0],
                        conf_ref[2 * p:2 * p + 2, 1])
        s = jax.nn.sigmoid(m).reshape(HH, W)
        sb = s.astype(jnp.bfloat16)
        v = _smooth_pair(sb, bh1_hi_ref, bh1_lo_ref, bh1_lo2_ref,
                         bh23_hi_ref, bw1_hi_ref, bw1_lo_ref, bw1_lo2_ref,
                         bw23_hi_ref, HH, W, rank)
        v = v.reshape(2, H, W)
        raw_ref[2 * p:2 * p + 2] = v
        mask = jnp.where(v > _THRESHOLD, 1.0, 0.0).astype(jnp.float32)
        t = jnp.sum(mask)
        total = t if total is None else total + t
        is_ego = ((pairs * pid + p) % ego_stride) == 0
        mask_ref[2 * p] = jnp.where(is_ego, jnp.ones_like(mask[0]), mask[0])
        mask_ref[2 * p + 1] = mask[1]
    sum_ref[0] = total[None, None]


def kernel(batch_confidence_maps, B):
    conf = batch_confidence_maps
    Bn, L, A, H, W = conf.shape
    n_maps = Bn * L
    flat = conf.reshape(n_maps, A, H, W)

    us, vs = _gauss_rank_factors()
    rank = len(us)
    bh_blocks = []
    bw_blocks = []
    for u, v in zip(us, vs):
        bh1 = _band_matrix(H, u)
        z = np.zeros_like(bh1)
        bh_blocks.append(np.block([[bh1, z], [z, bh1]]))
        bw_blocks.append(_band_matrix(W, v).T)
    bh = np.concatenate(bh_blocks, axis=0)
    bw = np.concatenate(bw_blocks, axis=0)

    bh_hi, bh_lo, bh_lo2 = _split_bf16(bh)
    bw_hi, bw_lo, bw_lo2 = _split_bf16(bw)
    HH = 2 * H
    bh1_hi, bh1_lo, bh1_lo2 = bh_hi[0:HH], bh_lo[0:HH], bh_lo2[0:HH]
    bh23_hi = bh_hi[HH:]
    bw1_hi, bw1_lo, bw1_lo2 = bw_hi[0:W], bw_lo[0:W], bw_lo2[0:W]
    bw23_hi = bw_hi[W:]

    pairs = 4
    n_steps = n_maps // (2 * pairs)
    const_spec = [
        pl.BlockSpec((HH, HH), lambda i: (0, 0)),
        pl.BlockSpec((HH, HH), lambda i: (0, 0)),
        pl.BlockSpec((HH, HH), lambda i: (0, 0)),
        pl.BlockSpec(((rank - 1) * HH, HH), lambda i: (0, 0)),
        pl.BlockSpec((W, W), lambda i: (0, 0)),
        pl.BlockSpec((W, W), lambda i: (0, 0)),
        pl.BlockSpec((W, W), lambda i: (0, 0)),
        pl.BlockSpec(((rank - 1) * W, W), lambda i: (0, 0)),
    ]
    raw, mask, sums = pl.pallas_call(
        functools.partial(_fused_kernel, ego_stride=L // 2, H=H, W=W,
                          rank=rank, pairs=pairs),
        grid=(n_steps,),
        in_specs=[pl.BlockSpec((2 * pairs, A, H, W), lambda i: (i, 0, 0, 0))]
                 + const_spec,
        out_specs=[
            pl.BlockSpec((2 * pairs, H, W), lambda i: (i, 0, 0)),
            pl.BlockSpec((2 * pairs, H, W), lambda i: (i, 0, 0)),
            pl.BlockSpec((1, 1, 1), lambda i: (i, 0, 0)),
        ],
        out_shape=[
            jax.ShapeDtypeStruct((n_maps, H, W), jnp.float32),
            jax.ShapeDtypeStruct((n_maps, H, W), jnp.float32),
            jax.ShapeDtypeStruct((n_steps, 1, 1), jnp.float32),
        ],
    )(flat, jnp.asarray(bh1_hi), jnp.asarray(bh1_lo), jnp.asarray(bh1_lo2),
      jnp.asarray(bh23_hi), jnp.asarray(bw1_hi), jnp.asarray(bw1_lo),
      jnp.asarray(bw1_lo2), jnp.asarray(bw23_hi))

    communication_masks = mask.reshape(n_maps, 1, H, W)
    raw_out = raw.reshape(Bn, L, 1, H, W)
    total = jnp.sum(sums)
    communication_rate = total / jnp.float32(L * H * W) / jnp.float32(Bn)
    batch_check = (jnp.asarray(B) - Bn) * 0
    communication_rate = communication_rate + batch_check.astype(jnp.float32)
    return (communication_masks, communication_rate, raw_out)


# speedup vs baseline: 13.4659x; 1.0002x over previous
import functools

import numpy as np
import jax
import jax.numpy as jnp
from jax.experimental import pallas as pl

_THRESHOLD = 0.5
_K_SIZE = 5
_C_SIGMA = 1.0


def _bf16_round(a):
    return np.asarray(a, dtype=np.float32).astype(jnp.bfloat16).astype(np.float32)


def _gauss_rank_factors(k_size=_K_SIZE, sigma=_C_SIGMA):
    center = k_size // 2
    x, y = np.mgrid[-center:k_size - center, -center:k_size - center]
    g = 1.0 / (2.0 * np.pi * sigma) * np.exp(
        -(np.square(x) + np.square(y)) / (2.0 * np.square(sigma)))
    wb = np.asarray(_bf16_round(g), dtype=np.float64)
    u_mat, s_vals, vt_mat = np.linalg.svd(wb)
    rank = int((s_vals > 1e-12).sum())
    us = [(u_mat[:, r] * s_vals[r]).astype(np.float32) for r in range(rank)]
    vs = [vt_mat[r].astype(np.float32) for r in range(rank)]
    return us, vs


def _band_matrix(n, taps):
    k = len(taps)
    c = k // 2
    m = np.zeros((n, n), dtype=np.float32)
    for d in range(k):
        off = d - c
        idx = np.arange(max(0, -off), min(n, n - off))
        m[idx, idx + off] = taps[d]
    return m


def _split_bf16(a32):
    hi = np.asarray(a32.astype(jnp.bfloat16))
    r1 = a32 - hi.astype(np.float32)
    lo = np.asarray(r1.astype(jnp.bfloat16))
    r2 = r1 - lo.astype(np.float32)
    lo2 = np.asarray(r2.astype(jnp.bfloat16))
    return hi, lo, lo2


def _dot16(a, b):
    return jax.lax.dot_general(
        a, b, (((1,), (0,)), ((), ())),
        preferred_element_type=jnp.float32)


def _smooth_pair(sb, bh1_hi_ref, bh1_lo_ref, bh1_lo2_ref, bh23_hi_ref,
                 bw1_hi_ref, bw1_lo_ref, bw1_lo2_ref, bw23_hi_ref,
                 HH, W, rank):
    x1 = (_dot16(bh1_hi_ref[...], sb) + _dot16(bh1_lo_ref[...], sb)
          + _dot16(bh1_lo2_ref[...], sb))
    a_hi = x1.astype(jnp.bfloat16)
    a_lo = (x1 - a_hi.astype(jnp.float32)).astype(jnp.bfloat16)
    v = (_dot16(a_hi, bw1_hi_ref[...]) + _dot16(a_hi, bw1_lo_ref[...])
         + _dot16(a_lo, bw1_hi_ref[...]) + _dot16(a_lo, bw1_lo_ref[...])
         + _dot16(a_hi, bw1_lo2_ref[...]))
    xs23 = _dot16(bh23_hi_ref[...], sb)
    for r in range(rank - 1):
        v = v + _dot16(xs23[r * HH:(r + 1) * HH].astype(jnp.bfloat16),
                       bw23_hi_ref[r * W:(r + 1) * W])
    return v


def _fused_kernel(conf_ref, bh1_hi_ref, bh1_lo_ref, bh1_lo2_ref, bh23_hi_ref,
                  bw1_hi_ref, bw1_lo_ref, bw1_lo2_ref, bw23_hi_ref,
                  raw_ref, mask_ref, sum_ref, *, ego_stride, H, W, rank,
                  pairs):
    pid = pl.program_id(0)
    HH = 2 * H
    total = None
    for p in range(pairs):
        m = jnp.maximum(conf_ref[2 * p:2 * p + 2, 0],
                        conf_ref[2 * p:2 * p + 2, 1])
        s = jax.nn.sigmoid(m).reshape(HH, W)
        sb = s.astype(jnp.bfloat16)
        v = _smooth_pair(sb, bh1_hi_ref, bh1_lo_ref, bh1_lo2_ref,
                         bh23_hi_ref, bw1_hi_ref, bw1_lo_ref, bw1_lo2_ref,
                         bw23_hi_ref, HH, W, rank)
        v = v.reshape(2, H, W)
        raw_ref[2 * p:2 * p + 2] = v
        mask = jnp.where(v > _THRESHOLD, 1.0, 0.0).astype(jnp.float32)
        t = jnp.sum(mask)
        total = t if total is None else total + t
        is_ego = ((pairs * pid + p) % ego_stride) == 0
        mask_ref[2 * p] = jnp.where(is_ego, jnp.ones_like(mask[0]), mask[0])
        mask_ref[2 * p + 1] = mask[1]
    sum_ref[0] = total[None, None]


def kernel(batch_confidence_maps, B):
    conf = batch_confidence_maps
    Bn, L, A, H, W = conf.shape
    n_maps = Bn * L
    flat = conf.reshape(n_maps, A, H, W)

    us, vs = _gauss_rank_factors()
    rank = len(us)
    bh_blocks = []
    bw_blocks = []
    for u, v in zip(us, vs):
        bh1 = _band_matrix(H, u)
        z = np.zeros_like(bh1)
        bh_blocks.append(np.block([[bh1, z], [z, bh1]]))
        bw_blocks.append(_band_matrix(W, v).T)
    bh = np.concatenate(bh_blocks, axis=0)
    bw = np.concatenate(bw_blocks, axis=0)

    bh_hi, bh_lo, bh_lo2 = _split_bf16(bh)
    bw_hi, bw_lo, bw_lo2 = _split_bf16(bw)
    HH = 2 * H
    bh1_hi, bh1_lo, bh1_lo2 = bh_hi[0:HH], bh_lo[0:HH], bh_lo2[0:HH]
    bh23_hi = bh_hi[HH:]
    bw1_hi, bw1_lo, bw1_lo2 = bw_hi[0:W], bw_lo[0:W], bw_lo2[0:W]
    bw23_hi = bw_hi[W:]

    pairs = 8
    n_steps = n_maps // (2 * pairs)
    const_spec = [
        pl.BlockSpec((HH, HH), lambda i: (0, 0)),
        pl.BlockSpec((HH, HH), lambda i: (0, 0)),
        pl.BlockSpec((HH, HH), lambda i: (0, 0)),
        pl.BlockSpec(((rank - 1) * HH, HH), lambda i: (0, 0)),
        pl.BlockSpec((W, W), lambda i: (0, 0)),
        pl.BlockSpec((W, W), lambda i: (0, 0)),
        pl.BlockSpec((W, W), lambda i: (0, 0)),
        pl.BlockSpec(((rank - 1) * W, W), lambda i: (0, 0)),
    ]
    raw, mask, sums = pl.pallas_call(
        functools.partial(_fused_kernel, ego_stride=L // 2, H=H, W=W,
                          rank=rank, pairs=pairs),
        grid=(n_steps,),
        in_specs=[pl.BlockSpec((2 * pairs, A, H, W), lambda i: (i, 0, 0, 0))]
                 + const_spec,
        out_specs=[
            pl.BlockSpec((2 * pairs, H, W), lambda i: (i, 0, 0)),
            pl.BlockSpec((2 * pairs, H, W), lambda i: (i, 0, 0)),
            pl.BlockSpec((1, 1, 1), lambda i: (i, 0, 0)),
        ],
        out_shape=[
            jax.ShapeDtypeStruct((n_maps, H, W), jnp.float32),
            jax.ShapeDtypeStruct((n_maps, H, W), jnp.float32),
            jax.ShapeDtypeStruct((n_steps, 1, 1), jnp.float32),
        ],
    )(flat, jnp.asarray(bh1_hi), jnp.asarray(bh1_lo), jnp.asarray(bh1_lo2),
      jnp.asarray(bh23_hi), jnp.asarray(bw1_hi), jnp.asarray(bw1_lo),
      jnp.asarray(bw1_lo2), jnp.asarray(bw23_hi))

    communication_masks = mask.reshape(n_maps, 1, H, W)
    raw_out = raw.reshape(Bn, L, 1, H, W)
    total = jnp.sum(sums)
    communication_rate = total / jnp.float32(L * H * W) / jnp.float32(Bn)
    batch_check = (jnp.asarray(B) - Bn) * 0
    communication_rate = communication_rate + batch_check.astype(jnp.float32)
    return (communication_masks, communication_rate, raw_out)


# speedup vs baseline: 13.4725x; 1.0005x over previous
import functools

import numpy as np
import jax
import jax.numpy as jnp
from jax.experimental import pallas as pl

_THRESHOLD = 0.5
_K_SIZE = 5
_C_SIGMA = 1.0


def _bf16_round(a):
    return np.asarray(a, dtype=np.float32).astype(jnp.bfloat16).astype(np.float32)


def _gauss_rank_factors(k_size=_K_SIZE, sigma=_C_SIGMA):
    center = k_size // 2
    x, y = np.mgrid[-center:k_size - center, -center:k_size - center]
    g = 1.0 / (2.0 * np.pi * sigma) * np.exp(
        -(np.square(x) + np.square(y)) / (2.0 * np.square(sigma)))
    wb = np.asarray(_bf16_round(g), dtype=np.float64)
    u_mat, s_vals, vt_mat = np.linalg.svd(wb)
    rank = int((s_vals > 1e-12).sum())
    us = [(u_mat[:, r] * s_vals[r]).astype(np.float32) for r in range(rank)]
    vs = [vt_mat[r].astype(np.float32) for r in range(rank)]
    return us, vs


def _band_matrix(n, taps):
    k = len(taps)
    c = k // 2
    m = np.zeros((n, n), dtype=np.float32)
    for d in range(k):
        off = d - c
        idx = np.arange(max(0, -off), min(n, n - off))
        m[idx, idx + off] = taps[d]
    return m


def _split_bf16(a32):
    hi = np.asarray(a32.astype(jnp.bfloat16))
    r1 = a32 - hi.astype(np.float32)
    lo = np.asarray(r1.astype(jnp.bfloat16))
    r2 = r1 - lo.astype(np.float32)
    lo2 = np.asarray(r2.astype(jnp.bfloat16))
    return hi, lo, lo2


def _dot16(a, b):
    return jax.lax.dot_general(
        a, b, (((1,), (0,)), ((), ())),
        preferred_element_type=jnp.float32)


def _smooth_pair(sb, bh1_hi_ref, bh1_lo_ref, bh1_lo2_ref, bh23_hi_ref,
                 bw1_hi_ref, bw1_lo_ref, bw1_lo2_ref, bw23_hi_ref,
                 HH, W, rank):
    x1 = (_dot16(bh1_hi_ref[...], sb) + _dot16(bh1_lo_ref[...], sb)
          + _dot16(bh1_lo2_ref[...], sb))
    a_hi = x1.astype(jnp.bfloat16)
    a_lo = (x1 - a_hi.astype(jnp.float32)).astype(jnp.bfloat16)
    v = (_dot16(a_hi, bw1_hi_ref[...]) + _dot16(a_hi, bw1_lo_ref[...])
         + _dot16(a_lo, bw1_hi_ref[...]) + _dot16(a_lo, bw1_lo_ref[...])
         + _dot16(a_hi, bw1_lo2_ref[...]))
    xs23 = _dot16(bh23_hi_ref[...], sb)
    for r in range(rank - 1):
        v = v + _dot16(xs23[r * HH:(r + 1) * HH].astype(jnp.bfloat16),
                       bw23_hi_ref[r * W:(r + 1) * W])
    return v


def _fused_kernel(conf_ref, bh1_hi_ref, bh1_lo_ref, bh1_lo2_ref, bh23_hi_ref,
                  bw1_hi_ref, bw1_lo_ref, bw1_lo2_ref, bw23_hi_ref,
                  raw_ref, mask_ref, sum_ref, *, ego_stride, H, W, rank,
                  pairs):
    pid = pl.program_id(0)
    HH = 2 * H
    total = None
    for p in range(pairs):
        m = jnp.maximum(conf_ref[2 * p:2 * p + 2, 0],
                        conf_ref[2 * p:2 * p + 2, 1])
        s = jax.nn.sigmoid(m).reshape(HH, W)
        sb = s.astype(jnp.bfloat16)
        v = _smooth_pair(sb, bh1_hi_ref, bh1_lo_ref, bh1_lo2_ref,
                         bh23_hi_ref, bw1_hi_ref, bw1_lo_ref, bw1_lo2_ref,
                         bw23_hi_ref, HH, W, rank)
        v = v.reshape(2, H, W)
        raw_ref[2 * p:2 * p + 2] = v
        mask = jnp.where(v > _THRESHOLD, 1.0, 0.0).astype(jnp.float32)
        t = jnp.sum(mask)
        total = t if total is None else total + t
        is_ego = ((pairs * pid + p) % ego_stride) == 0
        mask_ref[2 * p] = jnp.where(is_ego, jnp.ones_like(mask[0]), mask[0])
        mask_ref[2 * p + 1] = mask[1]
    sum_ref[0] = total[None, None]


def kernel(batch_confidence_maps, B):
    conf = batch_confidence_maps
    Bn, L, A, H, W = conf.shape
    n_maps = Bn * L
    flat = conf.reshape(n_maps, A, H, W)

    us, vs = _gauss_rank_factors()
    rank = len(us)
    bh_blocks = []
    bw_blocks = []
    for u, v in zip(us, vs):
        bh1 = _band_matrix(H, u)
        z = np.zeros_like(bh1)
        bh_blocks.append(np.block([[bh1, z], [z, bh1]]))
        bw_blocks.append(_band_matrix(W, v).T)
    bh = np.concatenate(bh_blocks, axis=0)
    bw = np.concatenate(bw_blocks, axis=0)

    bh_hi, bh_lo, bh_lo2 = _split_bf16(bh)
    bw_hi, bw_lo, bw_lo2 = _split_bf16(bw)
    HH = 2 * H
    bh1_hi, bh1_lo, bh1_lo2 = bh_hi[0:HH], bh_lo[0:HH], bh_lo2[0:HH]
    bh23_hi = bh_hi[HH:]
    bw1_hi, bw1_lo, bw1_lo2 = bw_hi[0:W], bw_lo[0:W], bw_lo2[0:W]
    bw23_hi = bw_hi[W:]

    pairs = 4
    n_steps = n_maps // (2 * pairs)
    const_spec = [
        pl.BlockSpec((HH, HH), lambda i: (0, 0)),
        pl.BlockSpec((HH, HH), lambda i: (0, 0)),
        pl.BlockSpec((HH, HH), lambda i: (0, 0)),
        pl.BlockSpec(((rank - 1) * HH, HH), lambda i: (0, 0)),
        pl.BlockSpec((W, W), lambda i: (0, 0)),
        pl.BlockSpec((W, W), lambda i: (0, 0)),
        pl.BlockSpec((W, W), lambda i: (0, 0)),
        pl.BlockSpec(((rank - 1) * W, W), lambda i: (0, 0)),
    ]
    raw, mask, sums = pl.pallas_call(
        functools.partial(_fused_kernel, ego_stride=L // 2, H=H, W=W,
                          rank=rank, pairs=pairs),
        grid=(n_steps,),
        in_specs=[pl.BlockSpec((2 * pairs, A, H, W), lambda i: (i, 0, 0, 0))]
                 + const_spec,
        out_specs=[
            pl.BlockSpec((2 * pairs, H, W), lambda i: (i, 0, 0)),
            pl.BlockSpec((2 * pairs, H, W), lambda i: (i, 0, 0)),
            pl.BlockSpec((1, 1, 1), lambda i: (i, 0, 0)),
        ],
        out_shape=[
            jax.ShapeDtypeStruct((n_maps, H, W), jnp.float32),
            jax.ShapeDtypeStruct((n_maps, H, W), jnp.float32),
            jax.ShapeDtypeStruct((n_steps, 1, 1), jnp.float32),
        ],
    )(flat, jnp.asarray(bh1_hi), jnp.asarray(bh1_lo), jnp.asarray(bh1_lo2),
      jnp.asarray(bh23_hi), jnp.asarray(bw1_hi), jnp.asarray(bw1_lo),
      jnp.asarray(bw1_lo2), jnp.asarray(bw23_hi))

    communication_masks = mask.reshape(n_maps, 1, H, W)
    raw_out = raw.reshape(Bn, L, 1, H, W)
    total = jnp.sum(sums)
    communication_rate = total / jnp.float32(L * H * W) / jnp.float32(Bn)
    batch_check = (jnp.asarray(B) - Bn) * 0
    communication_rate = communication_rate + batch_check.astype(jnp.float32)
    return (communication_masks, communication_rate, raw_out)
